# Initial kernel scaffold; baseline (speedup 1.0000x reference)
import functools

import jax
import jax.numpy as jnp
from jax import lax
from jax.experimental import pallas as pl
from jax.experimental.pallas import tpu as pltpu
from jax.experimental.pallas import tpu_sc as plsc

N = 10000
E = 320000
D = 128
H = 128
DIM = 3
OUT = 128

_NBLK = 1000
_EBLK = 2000


def _elu(v):
    return jnp.where(v > 0, v, jnp.exp(v) - 1.0)


def _pre_body(x_ref, posp_ref, wxs_ref, wxd_ref, wg_ref, be0_ref, a_ref, b_ref):
    g = jnp.dot(posp_ref[...], wg_ref[...], preferred_element_type=jnp.float32)
    xs = jnp.dot(x_ref[...], wxs_ref[...], preferred_element_type=jnp.float32)
    xd = jnp.dot(x_ref[...], wxd_ref[...], preferred_element_type=jnp.float32)
    a_ref[...] = xs - g + be0_ref[...]
    b_ref[...] = xd + g


def _node_tables(x, posp, wxs, wxd, wg, be0):
    grid = (N // _NBLK,)
    return pl.pallas_call(
        _pre_body,
        grid=grid,
        in_specs=[
            pl.BlockSpec((_NBLK, D), lambda i: (i, 0)),
            pl.BlockSpec((_NBLK, 8), lambda i: (i, 0)),
            pl.BlockSpec((D, H), lambda i: (0, 0)),
            pl.BlockSpec((D, H), lambda i: (0, 0)),
            pl.BlockSpec((8, H), lambda i: (0, 0)),
            pl.BlockSpec((1, H), lambda i: (0, 0)),
        ],
        out_specs=[
            pl.BlockSpec((_NBLK, H), lambda i: (i, 0)),
            pl.BlockSpec((_NBLK, H), lambda i: (i, 0)),
        ],
        out_shape=[
            jax.ShapeDtypeStruct((N, H), jnp.float32),
            jax.ShapeDtypeStruct((N, H), jnp.float32),
        ],
    )(x, posp, wxs, wxd, wg, be0)


def _edge_body(h1_ref, we2_ref, be2_ref, we3_ref, be3_ref, out_ref):
    h1 = _elu(h1_ref[...])
    h2 = _elu(jnp.dot(h1, we2_ref[...], preferred_element_type=jnp.float32)
              + be2_ref[...])
    out_ref[...] = (jnp.dot(h2, we3_ref[...], preferred_element_type=jnp.float32)
                    + be3_ref[...])


def _edge_mlp(h1pre, we2, be2, we3, be3):
    grid = (E // _EBLK,)
    return pl.pallas_call(
        _edge_body,
        grid=grid,
        in_specs=[
            pl.BlockSpec((_EBLK, H), lambda i: (i, 0)),
            pl.BlockSpec((H, H), lambda i: (0, 0)),
            pl.BlockSpec((1, H), lambda i: (0, 0)),
            pl.BlockSpec((H, H), lambda i: (0, 0)),
            pl.BlockSpec((1, H), lambda i: (0, 0)),
        ],
        out_specs=pl.BlockSpec((_EBLK, H), lambda i: (i, 0)),
        out_shape=jax.ShapeDtypeStruct((E, H), jnp.float32),
    )(h1pre, we2, be2, we3, be3)


def _node_body(x_ref, s0_ref, s1_ref, wn0x_ref, wn0e_ref, bn0_ref, wn2_ref,
               bn2_ref, wn3_ref, bn3_ref, wd0_ref, bd0_ref, wd2_ref, bd2_ref,
               wd3_ref, bd3_ref, out_ref):
    seg = s0_ref[...] + s1_ref[...]
    h = (jnp.dot(x_ref[...], wn0x_ref[...], preferred_element_type=jnp.float32)
         + jnp.dot(seg, wn0e_ref[...], preferred_element_type=jnp.float32)
         + bn0_ref[...])
    h = _elu(h)
    h = _elu(jnp.dot(h, wn2_ref[...], preferred_element_type=jnp.float32)
             + bn2_ref[...])
    na = jnp.dot(h, wn3_ref[...], preferred_element_type=jnp.float32) + bn3_ref[...]
    d = _elu(jnp.dot(na, wd0_ref[...], preferred_element_type=jnp.float32)
             + bd0_ref[...])
    d = _elu(jnp.dot(d, wd2_ref[...], preferred_element_type=jnp.float32)
             + bd2_ref[...])
    out_ref[...] = (jnp.dot(d, wd3_ref[...], preferred_element_type=jnp.float32)
                    + bd3_ref[...])


def _node_mlp(x, s0, s1, wn0x, wn0e, bn0, wn2, bn2, wn3, bn3,
              wd0, bd0, wd2, bd2, wd3, bd3):
    grid = (N // _NBLK,)
    row = lambda i: (i, 0)
    full = lambda i: (0, 0)
    return pl.pallas_call(
        _node_body,
        grid=grid,
        in_specs=[
            pl.BlockSpec((_NBLK, D), row),
            pl.BlockSpec((_NBLK, H), row),
            pl.BlockSpec((_NBLK, H), row),
            pl.BlockSpec((D, H), full),
            pl.BlockSpec((H, H), full),
            pl.BlockSpec((1, H), full),
            pl.BlockSpec((H, H), full),
            pl.BlockSpec((1, H), full),
            pl.BlockSpec((H, H), full),
            pl.BlockSpec((1, H), full),
            pl.BlockSpec((H, H), full),
            pl.BlockSpec((1, H), full),
            pl.BlockSpec((H, H), full),
            pl.BlockSpec((1, H), full),
            pl.BlockSpec((H, OUT), full),
            pl.BlockSpec((1, OUT), full),
        ],
        out_specs=pl.BlockSpec((_NBLK, OUT), row),
        out_shape=jax.ShapeDtypeStruct((N, OUT), jnp.float32),
    )(x, s0, s1, wn0x, wn0e, bn0, wn2, bn2, wn3, bn3,
      wd0, bd0, wd2, bd2, wd3, bd3)


def kernel(x, edge_index, pos, We0, be0, We2, be2, We3, be3,
           Wn0, bn0, Wn2, bn2, Wn3, bn3, Wd0, bd0, Wd2, bd2, Wd3, bd3):
    src = edge_index[0]
    dst = edge_index[1]
    posp = jnp.pad(pos, ((0, 0), (0, 8 - DIM)))
    wg = jnp.pad(We0[:DIM], ((0, 8 - DIM), (0, 0)))
    wxs = We0[DIM:DIM + D]
    wxd = We0[DIM + D:]
    a, b = _node_tables(x, posp, wxs, wxd, wg, be0.reshape(1, H))

    h1pre = a[src] + b[dst]
    edge_attr = _edge_mlp(h1pre, We2, be2.reshape(1, H), We3, be3.reshape(1, H))
    seg = jax.ops.segment_sum(edge_attr, src, num_segments=N)
    zeros = jnp.zeros((N, H), jnp.float32)

    return _node_mlp(x, seg, zeros, Wn0[:D], Wn0[D:], bn0.reshape(1, H),
                     Wn2, bn2.reshape(1, H), Wn3, bn3.reshape(1, H),
                     Wd0, bd0.reshape(1, H), Wd2, bd2.reshape(1, H),
                     Wd3, bd3.reshape(1, OUT))


# speedup vs baseline: 4.8589x; 4.8589x over previous
import functools

import jax
import jax.numpy as jnp
from jax import lax
from jax.experimental import pallas as pl
from jax.experimental.pallas import tpu as pltpu
from jax.experimental.pallas import tpu_sc as plsc

N = 10000
E = 320000
D = 128
H = 128
DIM = 3
OUT = 128

_NBLK = 1000
_EBLK = 2000


def _elu(v):
    return jnp.where(v > 0, v, jnp.exp(v) - 1.0)


def _pre_body(x_ref, posp_ref, wxs_ref, wxd_ref, wg_ref, be0_ref, a_ref, b_ref):
    g = jnp.dot(posp_ref[...], wg_ref[...], preferred_element_type=jnp.float32)
    xs = jnp.dot(x_ref[...], wxs_ref[...], preferred_element_type=jnp.float32)
    xd = jnp.dot(x_ref[...], wxd_ref[...], preferred_element_type=jnp.float32)
    a_ref[...] = xs - g + be0_ref[...]
    b_ref[...] = xd + g


def _node_tables(x, posp, wxs, wxd, wg, be0):
    grid = (N // _NBLK,)
    return pl.pallas_call(
        _pre_body,
        grid=grid,
        in_specs=[
            pl.BlockSpec((_NBLK, D), lambda i: (i, 0)),
            pl.BlockSpec((_NBLK, 8), lambda i: (i, 0)),
            pl.BlockSpec((D, H), lambda i: (0, 0)),
            pl.BlockSpec((D, H), lambda i: (0, 0)),
            pl.BlockSpec((8, H), lambda i: (0, 0)),
            pl.BlockSpec((1, H), lambda i: (0, 0)),
        ],
        out_specs=[
            pl.BlockSpec((_NBLK, H), lambda i: (i, 0)),
            pl.BlockSpec((_NBLK, H), lambda i: (i, 0)),
        ],
        out_shape=[
            jax.ShapeDtypeStruct((N, H), jnp.float32),
            jax.ShapeDtypeStruct((N, H), jnp.float32),
        ],
    )(x, posp, wxs, wxd, wg, be0)


def _edge_body(h1_ref, we2_ref, be2_ref, we3_ref, be3_ref, out_ref):
    h1 = _elu(h1_ref[...])
    h2 = _elu(jnp.dot(h1, we2_ref[...], preferred_element_type=jnp.float32)
              + be2_ref[...])
    out_ref[...] = (jnp.dot(h2, we3_ref[...], preferred_element_type=jnp.float32)
                    + be3_ref[...])


def _edge_mlp(h1pre, we2, be2, we3, be3):
    grid = (E // _EBLK,)
    return pl.pallas_call(
        _edge_body,
        grid=grid,
        in_specs=[
            pl.BlockSpec((_EBLK, H), lambda i: (i, 0)),
            pl.BlockSpec((H, H), lambda i: (0, 0)),
            pl.BlockSpec((1, H), lambda i: (0, 0)),
            pl.BlockSpec((H, H), lambda i: (0, 0)),
            pl.BlockSpec((1, H), lambda i: (0, 0)),
        ],
        out_specs=pl.BlockSpec((_EBLK, H), lambda i: (i, 0)),
        out_shape=jax.ShapeDtypeStruct((E, H), jnp.float32),
    )(h1pre, we2, be2, we3, be3)


def _node_body(x_ref, s0_ref, s1_ref, wn0x_ref, wn0e_ref, bn0_ref, wn2_ref,
               bn2_ref, wn3_ref, bn3_ref, wd0_ref, bd0_ref, wd2_ref, bd2_ref,
               wd3_ref, bd3_ref, out_ref):
    seg = s0_ref[...] + s1_ref[...]
    h = (jnp.dot(x_ref[...], wn0x_ref[...], preferred_element_type=jnp.float32)
         + jnp.dot(seg, wn0e_ref[...], preferred_element_type=jnp.float32)
         + bn0_ref[...])
    h = _elu(h)
    h = _elu(jnp.dot(h, wn2_ref[...], preferred_element_type=jnp.float32)
             + bn2_ref[...])
    na = jnp.dot(h, wn3_ref[...], preferred_element_type=jnp.float32) + bn3_ref[...]
    d = _elu(jnp.dot(na, wd0_ref[...], preferred_element_type=jnp.float32)
             + bd0_ref[...])
    d = _elu(jnp.dot(d, wd2_ref[...], preferred_element_type=jnp.float32)
             + bd2_ref[...])
    out_ref[...] = (jnp.dot(d, wd3_ref[...], preferred_element_type=jnp.float32)
                    + bd3_ref[...])


def _node_mlp(x, s0, s1, wn0x, wn0e, bn0, wn2, bn2, wn3, bn3,
              wd0, bd0, wd2, bd2, wd3, bd3):
    grid = (N // _NBLK,)
    row = lambda i: (i, 0)
    full = lambda i: (0, 0)
    return pl.pallas_call(
        _node_body,
        grid=grid,
        in_specs=[
            pl.BlockSpec((_NBLK, D), row),
            pl.BlockSpec((_NBLK, H), row),
            pl.BlockSpec((_NBLK, H), row),
            pl.BlockSpec((D, H), full),
            pl.BlockSpec((H, H), full),
            pl.BlockSpec((1, H), full),
            pl.BlockSpec((H, H), full),
            pl.BlockSpec((1, H), full),
            pl.BlockSpec((H, H), full),
            pl.BlockSpec((1, H), full),
            pl.BlockSpec((H, H), full),
            pl.BlockSpec((1, H), full),
            pl.BlockSpec((H, H), full),
            pl.BlockSpec((1, H), full),
            pl.BlockSpec((H, OUT), full),
            pl.BlockSpec((1, OUT), full),
        ],
        out_specs=pl.BlockSpec((_NBLK, OUT), row),
        out_shape=jax.ShapeDtypeStruct((N, OUT), jnp.float32),
    )(x, s0, s1, wn0x, wn0e, bn0, wn2, bn2, wn3, bn3,
      wd0, bd0, wd2, bd2, wd3, bd3)


_NC = 2
_NS = 16
_NW = _NC * _NS
_CH = 128
_NCHUNK = E // _CH
_NPAD = 10240
_SEG_PER_TILE = _NPAD // _NS

_sc_mesh = plsc.VectorSubcoreMesh(core_axis_name="c", subcore_axis_name="s")


@functools.partial(
    pl.kernel,
    out_type=jax.ShapeDtypeStruct((E, H), jnp.float32),
    mesh=_sc_mesh,
    scratch_types=[
        pltpu.VMEM((_CH,), jnp.int32),
        pltpu.VMEM((_CH,), jnp.int32),
        pltpu.VMEM((_CH, H), jnp.float32),
        pltpu.VMEM((_CH, H), jnp.float32),
        pltpu.SemaphoreType.DMA,
        pltpu.SemaphoreType.DMA,
    ],
)
def _sc_gather(a_hbm, b_hbm, src_hbm, dst_hbm, out_hbm,
               sidx, didx, bufa, bufb, sema, semb):
    wid = lax.axis_index("s") * _NC + lax.axis_index("c")
    nchunks_w = (_NCHUNK - wid + _NW - 1) // _NW

    def body(i, carry):
        c = wid + i * _NW
        base = c * _CH
        pltpu.sync_copy(src_hbm.at[pl.ds(base, _CH)], sidx)
        pltpu.sync_copy(dst_hbm.at[pl.ds(base, _CH)], didx)
        cpa = pltpu.async_copy(a_hbm.at[sidx], bufa, sema)
        cpb = pltpu.async_copy(b_hbm.at[didx], bufb, semb)
        cpa.wait()
        cpb.wait()

        def row(r, rc):
            for j in range(H // 16):
                va = bufa[r, pl.ds(j * 16, 16)]
                vb = bufb[r, pl.ds(j * 16, 16)]
                bufa[r, pl.ds(j * 16, 16)] = va + vb
            return rc

        lax.fori_loop(0, _CH, row, 0)
        pltpu.sync_copy(bufa, out_hbm.at[pl.ds(base, _CH)])
        return carry

    lax.fori_loop(0, nchunks_w, body, 0)


@functools.partial(
    pl.kernel,
    out_type=jax.ShapeDtypeStruct((_NC * _NPAD, H), jnp.float32),
    mesh=_sc_mesh,
    scratch_types=[
        pltpu.VMEM((_CH,), jnp.int32),
        pltpu.VMEM((_CH, H), jnp.float32),
        pltpu.VMEM_SHARED((_NPAD, H), jnp.float32),
    ],
)
def _sc_scatter(ea_hbm, src_hbm, zeros_hbm, out_hbm, sidx, buf, acc):
    c = lax.axis_index("c")
    s = lax.axis_index("s")
    wid = s * _NC + c
    pltpu.sync_copy(zeros_hbm.at[pl.ds(s * _SEG_PER_TILE, _SEG_PER_TILE)],
                    acc.at[pl.ds(s * _SEG_PER_TILE, _SEG_PER_TILE)])
    plsc.subcore_barrier()
    nchunks_w = (_NCHUNK - wid + _NW - 1) // _NW

    def body(i, carry):
        ch = wid + i * _NW
        base = ch * _CH
        pltpu.sync_copy(src_hbm.at[pl.ds(base, _CH)], sidx)
        pltpu.sync_copy(ea_hbm.at[pl.ds(base, _CH)], buf)
        pltpu.sync_copy(buf, acc.at[sidx], add=True)
        return carry

    lax.fori_loop(0, nchunks_w, body, 0)
    plsc.subcore_barrier()
    pltpu.sync_copy(acc.at[pl.ds(s * _SEG_PER_TILE, _SEG_PER_TILE)],
                    out_hbm.at[pl.ds(c * _NPAD + s * _SEG_PER_TILE, _SEG_PER_TILE)])


def kernel(x, edge_index, pos, We0, be0, We2, be2, We3, be3,
           Wn0, bn0, Wn2, bn2, Wn3, bn3, Wd0, bd0, Wd2, bd2, Wd3, bd3):
    src = edge_index[0]
    dst = edge_index[1]
    posp = jnp.pad(pos, ((0, 0), (0, 8 - DIM)))
    wg = jnp.pad(We0[:DIM], ((0, 8 - DIM), (0, 0)))
    wxs = We0[DIM:DIM + D]
    wxd = We0[DIM + D:]
    a, b = _node_tables(x, posp, wxs, wxd, wg, be0.reshape(1, H))

    h1pre = _sc_gather(a, b, src, dst)
    edge_attr = _edge_mlp(h1pre, We2, be2.reshape(1, H), We3, be3.reshape(1, H))
    seg2 = _sc_scatter(edge_attr, src, jnp.zeros((_NPAD, H), jnp.float32))

    return _node_mlp(x, seg2[:N], seg2[_NPAD:_NPAD + N], Wn0[:D], Wn0[D:], bn0.reshape(1, H),
                     Wn2, bn2.reshape(1, H), Wn3, bn3.reshape(1, H),
                     Wd0, bd0.reshape(1, H), Wd2, bd2.reshape(1, H),
                     Wd3, bd3.reshape(1, OUT))


# speedup vs baseline: 7.4619x; 1.5357x over previous
import functools

import jax
import jax.numpy as jnp
from jax import lax
from jax.experimental import pallas as pl
from jax.experimental.pallas import tpu as pltpu
from jax.experimental.pallas import tpu_sc as plsc

N = 10000
E = 320000
D = 128
H = 128
DIM = 3
OUT = 128

_NBLK = 1000
_EBLK = 2000


def _elu(v):
    return jnp.where(v > 0, v, jnp.exp(v) - 1.0)


def _pre_body(x_ref, posp_ref, wxs_ref, wxd_ref, wg_ref, be0_ref, a_ref, b_ref):
    g = jnp.dot(posp_ref[...], wg_ref[...], preferred_element_type=jnp.float32)
    xs = jnp.dot(x_ref[...], wxs_ref[...], preferred_element_type=jnp.float32)
    xd = jnp.dot(x_ref[...], wxd_ref[...], preferred_element_type=jnp.float32)
    a_ref[...] = xs - g + be0_ref[...]
    b_ref[...] = xd + g


def _node_tables(x, posp, wxs, wxd, wg, be0):
    grid = (N // _NBLK,)
    return pl.pallas_call(
        _pre_body,
        grid=grid,
        in_specs=[
            pl.BlockSpec((_NBLK, D), lambda i: (i, 0)),
            pl.BlockSpec((_NBLK, 8), lambda i: (i, 0)),
            pl.BlockSpec((D, H), lambda i: (0, 0)),
            pl.BlockSpec((D, H), lambda i: (0, 0)),
            pl.BlockSpec((8, H), lambda i: (0, 0)),
            pl.BlockSpec((1, H), lambda i: (0, 0)),
        ],
        out_specs=[
            pl.BlockSpec((_NBLK, H), lambda i: (i, 0)),
            pl.BlockSpec((_NBLK, H), lambda i: (i, 0)),
        ],
        out_shape=[
            jax.ShapeDtypeStruct((N, H), jnp.float32),
            jax.ShapeDtypeStruct((N, H), jnp.float32),
        ],
    )(x, posp, wxs, wxd, wg, be0)


def _edge_body(h1_ref, we2_ref, be2_ref, we3_ref, be3_ref, out_ref):
    h1 = _elu(h1_ref[...])
    h2 = _elu(jnp.dot(h1, we2_ref[...], preferred_element_type=jnp.float32)
              + be2_ref[...])
    out_ref[...] = (jnp.dot(h2, we3_ref[...], preferred_element_type=jnp.float32)
                    + be3_ref[...])


def _edge_mlp(h1pre, we2, be2, we3, be3):
    grid = (E // _EBLK,)
    return pl.pallas_call(
        _edge_body,
        grid=grid,
        in_specs=[
            pl.BlockSpec((_EBLK, H), lambda i: (i, 0)),
            pl.BlockSpec((H, H), lambda i: (0, 0)),
            pl.BlockSpec((1, H), lambda i: (0, 0)),
            pl.BlockSpec((H, H), lambda i: (0, 0)),
            pl.BlockSpec((1, H), lambda i: (0, 0)),
        ],
        out_specs=pl.BlockSpec((_EBLK, H), lambda i: (i, 0)),
        out_shape=jax.ShapeDtypeStruct((E, H), jnp.float32),
    )(h1pre, we2, be2, we3, be3)


def _node_body(x_ref, s0_ref, s1_ref, wn0x_ref, wn0e_ref, bn0_ref, wn2_ref,
               bn2_ref, wn3_ref, bn3_ref, wd0_ref, bd0_ref, wd2_ref, bd2_ref,
               wd3_ref, bd3_ref, out_ref):
    seg = s0_ref[...] + s1_ref[...]
    h = (jnp.dot(x_ref[...], wn0x_ref[...], preferred_element_type=jnp.float32)
         + jnp.dot(seg, wn0e_ref[...], preferred_element_type=jnp.float32)
         + bn0_ref[...])
    h = _elu(h)
    h = _elu(jnp.dot(h, wn2_ref[...], preferred_element_type=jnp.float32)
             + bn2_ref[...])
    na = jnp.dot(h, wn3_ref[...], preferred_element_type=jnp.float32) + bn3_ref[...]
    d = _elu(jnp.dot(na, wd0_ref[...], preferred_element_type=jnp.float32)
             + bd0_ref[...])
    d = _elu(jnp.dot(d, wd2_ref[...], preferred_element_type=jnp.float32)
             + bd2_ref[...])
    out_ref[...] = (jnp.dot(d, wd3_ref[...], preferred_element_type=jnp.float32)
                    + bd3_ref[...])


def _node_mlp(x, s0, s1, wn0x, wn0e, bn0, wn2, bn2, wn3, bn3,
              wd0, bd0, wd2, bd2, wd3, bd3):
    grid = (N // _NBLK,)
    row = lambda i: (i, 0)
    full = lambda i: (0, 0)
    return pl.pallas_call(
        _node_body,
        grid=grid,
        in_specs=[
            pl.BlockSpec((_NBLK, D), row),
            pl.BlockSpec((_NBLK, H), row),
            pl.BlockSpec((_NBLK, H), row),
            pl.BlockSpec((D, H), full),
            pl.BlockSpec((H, H), full),
            pl.BlockSpec((1, H), full),
            pl.BlockSpec((H, H), full),
            pl.BlockSpec((1, H), full),
            pl.BlockSpec((H, H), full),
            pl.BlockSpec((1, H), full),
            pl.BlockSpec((H, H), full),
            pl.BlockSpec((1, H), full),
            pl.BlockSpec((H, H), full),
            pl.BlockSpec((1, H), full),
            pl.BlockSpec((H, OUT), full),
            pl.BlockSpec((1, OUT), full),
        ],
        out_specs=pl.BlockSpec((_NBLK, OUT), row),
        out_shape=jax.ShapeDtypeStruct((N, OUT), jnp.float32),
    )(x, s0, s1, wn0x, wn0e, bn0, wn2, bn2, wn3, bn3,
      wd0, bd0, wd2, bd2, wd3, bd3)


_NC = 2
_NS = 16
_NW = _NC * _NS
_CH = 128
_NCHUNK = E // _CH
_NPAD = 10240
_SEG_PER_TILE = _NPAD // _NS

_sc_mesh = plsc.VectorSubcoreMesh(core_axis_name="c", subcore_axis_name="s")


_WCH = _NCHUNK // _NW
_TAIL0 = _NW * _WCH
_NTAIL = _NCHUNK - _TAIL0


@functools.partial(
    pl.kernel,
    out_type=jax.ShapeDtypeStruct((E, H), jnp.float32),
    mesh=_sc_mesh,
    scratch_types=[
        pltpu.VMEM((2, _CH), jnp.int32),
        pltpu.VMEM((2, _CH), jnp.int32),
        pltpu.VMEM((2, _CH, H), jnp.float32),
        pltpu.VMEM((2, _CH, H), jnp.float32),
        pltpu.SemaphoreType.DMA,
        pltpu.SemaphoreType.DMA,
        pltpu.SemaphoreType.DMA,
        pltpu.SemaphoreType.DMA,
        pltpu.SemaphoreType.DMA,
        pltpu.SemaphoreType.DMA,
    ],
)
def _sc_gather(a_hbm, b_hbm, src_hbm, dst_hbm, out_hbm,
               sidx, didx, bufa, bufb,
               isem0, isem1, rsem0, rsem1, osem0, osem1):
    wid = lax.axis_index("s") * _NC + lax.axis_index("c")
    strip = wid * _WCH * _CH
    isems = (isem0, isem1)
    rsems = (rsem0, rsem1)
    osems = (osem0, osem1)

    def issue_idx(c, s, sem):
        base = strip + c * _CH
        pltpu.async_copy(src_hbm.at[pl.ds(base, _CH)], sidx.at[s], sem)
        pltpu.async_copy(dst_hbm.at[pl.ds(base, _CH)], didx.at[s], sem)

    def wait_idx(s, sem):
        pltpu.make_async_copy(src_hbm.at[pl.ds(0, _CH)], sidx.at[s], sem).wait()
        pltpu.make_async_copy(dst_hbm.at[pl.ds(0, _CH)], didx.at[s], sem).wait()

    def issue_rows(s, sem):
        pltpu.async_copy(a_hbm.at[sidx.at[s]], bufa.at[s], sem)
        pltpu.async_copy(b_hbm.at[didx.at[s]], bufb.at[s], sem)

    def wait_rows(s, sem):
        pltpu.make_async_copy(a_hbm.at[pl.ds(0, _CH)], bufa.at[s], sem).wait()
        pltpu.make_async_copy(b_hbm.at[pl.ds(0, _CH)], bufb.at[s], sem).wait()

    def wait_out(s, sem):
        pltpu.make_async_copy(bufa.at[s], out_hbm.at[pl.ds(0, _CH)], sem).wait()

    def compute_add(s):
        def row(r, rc):
            for j in range(H // 16):
                va = bufa[s, r, pl.ds(j * 16, 16)]
                vb = bufb[s, r, pl.ds(j * 16, 16)]
                bufa[s, r, pl.ds(j * 16, 16)] = va + vb
            return rc
        lax.fori_loop(0, _CH, row, 0)

    issue_idx(0, 0, isems[0])
    issue_idx(1, 1, isems[1])
    wait_idx(0, isems[0])
    issue_rows(0, rsems[0])

    def step(i, s, pf_rows, drain_out, pf_idx):
        sp = 1 - s

        @pl.when(pf_rows)
        def _prefetch_rows():
            wait_idx(sp, isems[sp])

            @pl.when(drain_out)
            def _drain_out():
                wait_out(sp, osems[sp])

            issue_rows(sp, rsems[sp])

        wait_rows(s, rsems[s])

        @pl.when(pf_idx)
        def _prefetch_idx():
            issue_idx(i + 2, s, isems[s])

        compute_add(s)
        base = strip + i * _CH
        pltpu.async_copy(bufa.at[s], out_hbm.at[pl.ds(base, _CH)], osems[s])

    def body(k, carry):
        i0 = 2 * k
        step(i0, 0, i0 + 1 < _WCH, k >= 1, i0 + 2 < _WCH)
        i1 = 2 * k + 1
        step(i1, 1, i1 + 1 < _WCH, i1 >= 1, i1 + 2 < _WCH)
        return carry

    lax.fori_loop(0, _WCH // 2, body, 0)
    wait_out(0, osems[0])
    wait_out(1, osems[1])

    @pl.when(wid < _NTAIL)
    def _tail():
        base = (_TAIL0 + wid) * _CH
        pltpu.sync_copy(src_hbm.at[pl.ds(base, _CH)], sidx.at[0])
        pltpu.sync_copy(dst_hbm.at[pl.ds(base, _CH)], didx.at[0])
        cpa = pltpu.async_copy(a_hbm.at[sidx.at[0]], bufa.at[0], rsem0)
        cpb = pltpu.async_copy(b_hbm.at[didx.at[0]], bufb.at[0], rsem1)
        cpa.wait()
        cpb.wait()
        compute_add(0)
        pltpu.sync_copy(bufa.at[0], out_hbm.at[pl.ds(base, _CH)])


@functools.partial(
    pl.kernel,
    out_type=jax.ShapeDtypeStruct((_NC * _NPAD, H), jnp.float32),
    mesh=_sc_mesh,
    scratch_types=[
        pltpu.VMEM((2, _CH), jnp.int32),
        pltpu.VMEM((2, _CH, H), jnp.float32),
        pltpu.VMEM_SHARED((_NPAD, H), jnp.float32),
        pltpu.SemaphoreType.DMA,
        pltpu.SemaphoreType.DMA,
        pltpu.SemaphoreType.DMA,
        pltpu.SemaphoreType.DMA,
    ],
)
def _sc_scatter(ea_hbm, src_hbm, zeros_hbm, out_hbm,
                sidx, buf, acc, isem0, isem1, dsem0, dsem1):
    cax = lax.axis_index("c")
    sax = lax.axis_index("s")
    wid = sax * _NC + cax
    strip = wid * _WCH * _CH
    pltpu.sync_copy(zeros_hbm.at[pl.ds(sax * _SEG_PER_TILE, _SEG_PER_TILE)],
                    acc.at[pl.ds(sax * _SEG_PER_TILE, _SEG_PER_TILE)])
    plsc.subcore_barrier()

    isems = (isem0, isem1)
    dsems = (dsem0, dsem1)

    def issue(c, s):
        base = strip + c * _CH
        pltpu.async_copy(src_hbm.at[pl.ds(base, _CH)], sidx.at[s], isems[s])
        pltpu.async_copy(ea_hbm.at[pl.ds(base, _CH)], buf.at[s], dsems[s])

    def wait_in(s):
        pltpu.make_async_copy(src_hbm.at[pl.ds(0, _CH)], sidx.at[s],
                              isems[s]).wait()
        pltpu.make_async_copy(ea_hbm.at[pl.ds(0, _CH)], buf.at[s],
                              dsems[s]).wait()

    issue(0, 0)

    def step(i, s, prefetch):
        sp = 1 - s

        @pl.when(prefetch)
        def _prefetch():
            issue(i + 1, sp)

        wait_in(s)
        pltpu.sync_copy(buf.at[s], acc.at[sidx.at[s]], add=True)

    def body(k, carry):
        i0 = 2 * k
        step(i0, 0, i0 + 1 < _WCH)
        i1 = 2 * k + 1
        step(i1, 1, i1 + 1 < _WCH)
        return carry

    lax.fori_loop(0, _WCH // 2, body, 0)

    @pl.when(wid < _NTAIL)
    def _tail():
        base = (_TAIL0 + wid) * _CH
        pltpu.sync_copy(src_hbm.at[pl.ds(base, _CH)], sidx.at[0])
        pltpu.sync_copy(ea_hbm.at[pl.ds(base, _CH)], buf.at[0])
        pltpu.sync_copy(buf.at[0], acc.at[sidx.at[0]], add=True)

    plsc.subcore_barrier()
    pltpu.sync_copy(acc.at[pl.ds(sax * _SEG_PER_TILE, _SEG_PER_TILE)],
                    out_hbm.at[pl.ds(cax * _NPAD + sax * _SEG_PER_TILE,
                                     _SEG_PER_TILE)])


def kernel(x, edge_index, pos, We0, be0, We2, be2, We3, be3,
           Wn0, bn0, Wn2, bn2, Wn3, bn3, Wd0, bd0, Wd2, bd2, Wd3, bd3):
    src = edge_index[0]
    dst = edge_index[1]
    posp = jnp.pad(pos, ((0, 0), (0, 8 - DIM)))
    wg = jnp.pad(We0[:DIM], ((0, 8 - DIM), (0, 0)))
    wxs = We0[DIM:DIM + D]
    wxd = We0[DIM + D:]
    a, b = _node_tables(x, posp, wxs, wxd, wg, be0.reshape(1, H))

    h1pre = _sc_gather(a, b, src, dst)
    edge_attr = _edge_mlp(h1pre, We2, be2.reshape(1, H), We3, be3.reshape(1, H))
    seg2 = _sc_scatter(edge_attr, src, jnp.zeros((_NPAD, H), jnp.float32))

    return _node_mlp(x, seg2[:N], seg2[_NPAD:_NPAD + N], Wn0[:D], Wn0[D:], bn0.reshape(1, H),
                     Wn2, bn2.reshape(1, H), Wn3, bn3.reshape(1, H),
                     Wd0, bd0.reshape(1, H), Wd2, bd2.reshape(1, H),
                     Wd3, bd3.reshape(1, OUT))


# speedup vs baseline: 8.5532x; 1.1462x over previous
import functools

import jax
import jax.numpy as jnp
from jax import lax
from jax.experimental import pallas as pl
from jax.experimental.pallas import tpu as pltpu
from jax.experimental.pallas import tpu_sc as plsc

N = 10000
E = 320000
D = 128
H = 128
DIM = 3
OUT = 128

_NBLK = 1000
_EBLK = 2000


def _elu(v):
    return jnp.where(v > 0, v, jnp.exp(v) - 1.0)


def _pre_body(x_ref, posp_ref, wxs_ref, wxd_ref, wg_ref, be0_ref, a_ref, b_ref):
    g = jnp.dot(posp_ref[...], wg_ref[...], preferred_element_type=jnp.float32)
    xs = jnp.dot(x_ref[...], wxs_ref[...], preferred_element_type=jnp.float32)
    xd = jnp.dot(x_ref[...], wxd_ref[...], preferred_element_type=jnp.float32)
    a_ref[...] = xs - g + be0_ref[...]
    b_ref[...] = xd + g


def _node_tables(x, posp, wxs, wxd, wg, be0):
    grid = (N // _NBLK,)
    return pl.pallas_call(
        _pre_body,
        grid=grid,
        in_specs=[
            pl.BlockSpec((_NBLK, D), lambda i: (i, 0)),
            pl.BlockSpec((_NBLK, 8), lambda i: (i, 0)),
            pl.BlockSpec((D, H), lambda i: (0, 0)),
            pl.BlockSpec((D, H), lambda i: (0, 0)),
            pl.BlockSpec((8, H), lambda i: (0, 0)),
            pl.BlockSpec((1, H), lambda i: (0, 0)),
        ],
        out_specs=[
            pl.BlockSpec((_NBLK, H), lambda i: (i, 0)),
            pl.BlockSpec((_NBLK, H), lambda i: (i, 0)),
        ],
        out_shape=[
            jax.ShapeDtypeStruct((N, H), jnp.float32),
            jax.ShapeDtypeStruct((N, H), jnp.float32),
        ],
    )(x, posp, wxs, wxd, wg, be0)


def _edge_body(h1_ref, we2_ref, be2_ref, we3_ref, be3_ref, out_ref):
    h1 = _elu(h1_ref[...])
    h2 = _elu(jnp.dot(h1, we2_ref[...], preferred_element_type=jnp.float32)
              + be2_ref[...])
    out_ref[...] = (jnp.dot(h2, we3_ref[...], preferred_element_type=jnp.float32)
                    + be3_ref[...])


def _make_edge_mlp(ne):
    grid = (ne // _EBLK,)

    def call(h1pre, we2, be2, we3, be3):
        return pl.pallas_call(
            _edge_body,
            grid=grid,
            in_specs=[
                pl.BlockSpec((_EBLK, H), lambda i: (i, 0)),
                pl.BlockSpec((H, H), lambda i: (0, 0)),
                pl.BlockSpec((1, H), lambda i: (0, 0)),
                pl.BlockSpec((H, H), lambda i: (0, 0)),
                pl.BlockSpec((1, H), lambda i: (0, 0)),
            ],
            out_specs=pl.BlockSpec((_EBLK, H), lambda i: (i, 0)),
            out_shape=jax.ShapeDtypeStruct((ne, H), jnp.float32),
        )(h1pre, we2, be2, we3, be3)

    return call


def _node_body(x_ref, s0_ref, s1_ref, s2_ref, s3_ref, wn0x_ref, wn0e_ref,
               bn0_ref, wn2_ref, bn2_ref, wn3_ref, bn3_ref, wd0_ref, bd0_ref,
               wd2_ref, bd2_ref, wd3_ref, bd3_ref, out_ref):
    seg = (s0_ref[...] + s1_ref[...]) + (s2_ref[...] + s3_ref[...])
    h = (jnp.dot(x_ref[...], wn0x_ref[...], preferred_element_type=jnp.float32)
         + jnp.dot(seg, wn0e_ref[...], preferred_element_type=jnp.float32)
         + bn0_ref[...])
    h = _elu(h)
    h = _elu(jnp.dot(h, wn2_ref[...], preferred_element_type=jnp.float32)
             + bn2_ref[...])
    na = jnp.dot(h, wn3_ref[...], preferred_element_type=jnp.float32) + bn3_ref[...]
    d = _elu(jnp.dot(na, wd0_ref[...], preferred_element_type=jnp.float32)
             + bd0_ref[...])
    d = _elu(jnp.dot(d, wd2_ref[...], preferred_element_type=jnp.float32)
             + bd2_ref[...])
    out_ref[...] = (jnp.dot(d, wd3_ref[...], preferred_element_type=jnp.float32)
                    + bd3_ref[...])


def _node_mlp(x, s0, s1, s2, s3, wn0x, wn0e, bn0, wn2, bn2, wn3, bn3,
              wd0, bd0, wd2, bd2, wd3, bd3):
    grid = (N // _NBLK,)
    row = lambda i: (i, 0)
    full = lambda i: (0, 0)
    return pl.pallas_call(
        _node_body,
        grid=grid,
        in_specs=[
            pl.BlockSpec((_NBLK, D), row),
            pl.BlockSpec((_NBLK, H), row),
            pl.BlockSpec((_NBLK, H), row),
            pl.BlockSpec((_NBLK, H), row),
            pl.BlockSpec((_NBLK, H), row),
            pl.BlockSpec((D, H), full),
            pl.BlockSpec((H, H), full),
            pl.BlockSpec((1, H), full),
            pl.BlockSpec((H, H), full),
            pl.BlockSpec((1, H), full),
            pl.BlockSpec((H, H), full),
            pl.BlockSpec((1, H), full),
            pl.BlockSpec((H, H), full),
            pl.BlockSpec((1, H), full),
            pl.BlockSpec((H, H), full),
            pl.BlockSpec((1, H), full),
            pl.BlockSpec((H, OUT), full),
            pl.BlockSpec((1, OUT), full),
        ],
        out_specs=pl.BlockSpec((_NBLK, OUT), row),
        out_shape=jax.ShapeDtypeStruct((N, OUT), jnp.float32),
    )(x, s0, s1, s2, s3, wn0x, wn0e, bn0, wn2, bn2, wn3, bn3,
      wd0, bd0, wd2, bd2, wd3, bd3)


_NC = 2
_NS = 16
_NW = _NC * _NS
_CH = 128
_NPAD = 10240
_SEG_PER_TILE = _NPAD // _NS

_sc_mesh = plsc.VectorSubcoreMesh(core_axis_name="c", subcore_axis_name="s")


def _make_sc_gather(ne):
    nchunk = ne // _CH
    wch = nchunk // _NW
    ntail = nchunk - _NW * wch
    tail0 = _NW * wch
    npairs = (wch + 1) // 2
    odd = (wch % 2) == 1

    @functools.partial(
        pl.kernel,
        out_type=jax.ShapeDtypeStruct((ne, H), jnp.float32),
        mesh=_sc_mesh,
        scratch_types=[
            pltpu.VMEM((2, _CH), jnp.int32),
            pltpu.VMEM((2, _CH), jnp.int32),
            pltpu.VMEM((2, _CH, H), jnp.float32),
            pltpu.VMEM((2, _CH, H), jnp.float32),
            pltpu.SemaphoreType.DMA,
            pltpu.SemaphoreType.DMA,
            pltpu.SemaphoreType.DMA,
            pltpu.SemaphoreType.DMA,
            pltpu.SemaphoreType.DMA,
            pltpu.SemaphoreType.DMA,
        ],
    )
    def gather(a_hbm, b_hbm, src_hbm, dst_hbm, out_hbm,
               sidx, didx, bufa, bufb,
               isem0, isem1, rsem0, rsem1, osem0, osem1):
        wid = lax.axis_index("s") * _NC + lax.axis_index("c")
        strip = wid * wch * _CH
        isems = (isem0, isem1)
        rsems = (rsem0, rsem1)
        osems = (osem0, osem1)

        def issue_idx(c, s, sem):
            base = strip + c * _CH
            pltpu.async_copy(src_hbm.at[pl.ds(base, _CH)], sidx.at[s], sem)
            pltpu.async_copy(dst_hbm.at[pl.ds(base, _CH)], didx.at[s], sem)

        def wait_idx(s, sem):
            pltpu.make_async_copy(src_hbm.at[pl.ds(0, _CH)], sidx.at[s],
                                  sem).wait()
            pltpu.make_async_copy(dst_hbm.at[pl.ds(0, _CH)], didx.at[s],
                                  sem).wait()

        def issue_rows(s, sem):
            pltpu.async_copy(a_hbm.at[sidx.at[s]], bufa.at[s], sem)
            pltpu.async_copy(b_hbm.at[didx.at[s]], bufb.at[s], sem)

        def wait_rows(s, sem):
            pltpu.make_async_copy(a_hbm.at[pl.ds(0, _CH)], bufa.at[s],
                                  sem).wait()
            pltpu.make_async_copy(b_hbm.at[pl.ds(0, _CH)], bufb.at[s],
                                  sem).wait()

        def wait_out(s, sem):
            pltpu.make_async_copy(bufa.at[s], out_hbm.at[pl.ds(0, _CH)],
                                  sem).wait()

        def compute_add(s):
            def row(r, rc):
                for j in range(H // 16):
                    va = bufa[s, r, pl.ds(j * 16, 16)]
                    vb = bufb[s, r, pl.ds(j * 16, 16)]
                    bufa[s, r, pl.ds(j * 16, 16)] = va + vb
                return rc
            lax.fori_loop(0, _CH, row, 0)

        issue_idx(0, 0, isems[0])
        issue_idx(1, 1, isems[1])
        wait_idx(0, isems[0])
        issue_rows(0, rsems[0])

        def step(i, s, pf_rows, drain_out, pf_idx):
            sp = 1 - s

            @pl.when(pf_rows)
            def _prefetch_rows():
                wait_idx(sp, isems[sp])

                @pl.when(drain_out)
                def _drain_out():
                    wait_out(sp, osems[sp])

                issue_rows(sp, rsems[sp])

            wait_rows(s, rsems[s])

            @pl.when(pf_idx)
            def _prefetch_idx():
                issue_idx(i + 2, s, isems[s])

            compute_add(s)
            base = strip + i * _CH
            pltpu.async_copy(bufa.at[s], out_hbm.at[pl.ds(base, _CH)],
                             osems[s])

        def body(k, carry):
            i0 = 2 * k
            step(i0, 0, i0 + 1 < wch, k >= 1, i0 + 2 < wch)
            i1 = 2 * k + 1
            if odd:
                @pl.when(i1 < wch)
                def _odd_step():
                    step(i1, 1, i1 + 1 < wch, i1 >= 1, i1 + 2 < wch)
            else:
                step(i1, 1, i1 + 1 < wch, i1 >= 1, i1 + 2 < wch)
            return carry

        lax.fori_loop(0, npairs, body, 0)
        wait_out(0, osems[0])
        wait_out(1, osems[1])

        @pl.when(wid < ntail)
        def _tail():
            base = (tail0 + wid) * _CH
            pltpu.sync_copy(src_hbm.at[pl.ds(base, _CH)], sidx.at[0])
            pltpu.sync_copy(dst_hbm.at[pl.ds(base, _CH)], didx.at[0])
            cpa = pltpu.async_copy(a_hbm.at[sidx.at[0]], bufa.at[0], rsem0)
            cpb = pltpu.async_copy(b_hbm.at[didx.at[0]], bufb.at[0], rsem1)
            cpa.wait()
            cpb.wait()
            compute_add(0)
            pltpu.sync_copy(bufa.at[0], out_hbm.at[pl.ds(base, _CH)])

    return gather


def _make_sc_scatter(ne):
    nchunk = ne // _CH
    wch = nchunk // _NW
    ntail = nchunk - _NW * wch
    tail0 = _NW * wch
    npairs = (wch + 1) // 2
    odd = (wch % 2) == 1

    @functools.partial(
        pl.kernel,
        out_type=jax.ShapeDtypeStruct((_NC * _NPAD, H), jnp.float32),
        mesh=_sc_mesh,
        scratch_types=[
            pltpu.VMEM((2, _CH), jnp.int32),
            pltpu.VMEM((2, _CH, H), jnp.float32),
            pltpu.VMEM_SHARED((_NPAD, H), jnp.float32),
            pltpu.SemaphoreType.DMA,
            pltpu.SemaphoreType.DMA,
            pltpu.SemaphoreType.DMA,
            pltpu.SemaphoreType.DMA,
        ],
    )
    def scatter(ea_hbm, src_hbm, zeros_hbm, out_hbm,
                sidx, buf, acc, isem0, isem1, dsem0, dsem1):
        cax = lax.axis_index("c")
        sax = lax.axis_index("s")
        wid = sax * _NC + cax
        strip = wid * wch * _CH
        pltpu.sync_copy(zeros_hbm.at[pl.ds(sax * _SEG_PER_TILE, _SEG_PER_TILE)],
                        acc.at[pl.ds(sax * _SEG_PER_TILE, _SEG_PER_TILE)])
        plsc.subcore_barrier()

        isems = (isem0, isem1)
        dsems = (dsem0, dsem1)

        def issue(c, s):
            base = strip + c * _CH
            pltpu.async_copy(src_hbm.at[pl.ds(base, _CH)], sidx.at[s],
                             isems[s])
            pltpu.async_copy(ea_hbm.at[pl.ds(base, _CH)], buf.at[s], dsems[s])

        def wait_in(s):
            pltpu.make_async_copy(src_hbm.at[pl.ds(0, _CH)], sidx.at[s],
                                  isems[s]).wait()
            pltpu.make_async_copy(ea_hbm.at[pl.ds(0, _CH)], buf.at[s],
                                  dsems[s]).wait()

        issue(0, 0)

        def step(i, s, prefetch):
            sp = 1 - s

            @pl.when(prefetch)
            def _prefetch():
                issue(i + 1, sp)

            wait_in(s)
            pltpu.sync_copy(buf.at[s], acc.at[sidx.at[s]], add=True)

        def body(k, carry):
            i0 = 2 * k
            step(i0, 0, i0 + 1 < wch)
            i1 = 2 * k + 1
            if odd:
                @pl.when(i1 < wch)
                def _odd_step():
                    step(i1, 1, i1 + 1 < wch)
            else:
                step(i1, 1, i1 + 1 < wch)
            return carry

        lax.fori_loop(0, npairs, body, 0)

        @pl.when(wid < ntail)
        def _tail():
            base = (tail0 + wid) * _CH
            pltpu.sync_copy(src_hbm.at[pl.ds(base, _CH)], sidx.at[0])
            pltpu.sync_copy(ea_hbm.at[pl.ds(base, _CH)], buf.at[0])
            pltpu.sync_copy(buf.at[0], acc.at[sidx.at[0]], add=True)

        plsc.subcore_barrier()
        pltpu.sync_copy(acc.at[pl.ds(sax * _SEG_PER_TILE, _SEG_PER_TILE)],
                        out_hbm.at[pl.ds(cax * _NPAD + sax * _SEG_PER_TILE,
                                         _SEG_PER_TILE)])

    return scatter


_EHALF = E // 2
_sc_gather_half = _make_sc_gather(_EHALF)
_sc_scatter_half = _make_sc_scatter(_EHALF)
_edge_mlp_half = _make_edge_mlp(_EHALF)


def kernel(x, edge_index, pos, We0, be0, We2, be2, We3, be3,
           Wn0, bn0, Wn2, bn2, Wn3, bn3, Wd0, bd0, Wd2, bd2, Wd3, bd3):
    src = edge_index[0]
    dst = edge_index[1]
    posp = jnp.pad(pos, ((0, 0), (0, 8 - DIM)))
    wg = jnp.pad(We0[:DIM], ((0, 8 - DIM), (0, 0)))
    wxs = We0[DIM:DIM + D]
    wxd = We0[DIM + D:]
    a, b = _node_tables(x, posp, wxs, wxd, wg, be0.reshape(1, H))

    src0, src1 = src[:_EHALF], src[_EHALF:]
    dst0, dst1 = dst[:_EHALF], dst[_EHALF:]
    h10 = _sc_gather_half(a, b, src0, dst0)
    h11 = _sc_gather_half(a, b, src1, dst1)
    be2r, be3r = be2.reshape(1, H), be3.reshape(1, H)
    ea0 = _edge_mlp_half(h10, We2, be2r, We3, be3r)
    ea1 = _edge_mlp_half(h11, We2, be2r, We3, be3r)
    zeros = jnp.zeros((_NPAD, H), jnp.float32)
    sg0 = _sc_scatter_half(ea0, src0, zeros)
    sg1 = _sc_scatter_half(ea1, src1, zeros)

    return _node_mlp(x, sg0[:N], sg0[_NPAD:_NPAD + N],
                     sg1[:N], sg1[_NPAD:_NPAD + N],
                     Wn0[:D], Wn0[D:], bn0.reshape(1, H),
                     Wn2, bn2.reshape(1, H), Wn3, bn3.reshape(1, H),
                     Wd0, bd0.reshape(1, H), Wd2, bd2.reshape(1, H),
                     Wd3, bd3.reshape(1, OUT))
